# baseline (speedup 1.0000x reference)
import jax
import jax.numpy as jnp
from jax import lax
from jax.experimental import pallas as pl
from jax.experimental.pallas import tpu as pltpu

_NUM_MASKS = 512
_N_POINTS = 20000
_ALPHA = 0.25

_PT_BLOCK = 2000
_GRID = _N_POINTS // _PT_BLOCK
_RG = 8
_UNROLL = 5
_NCHUNK = _PT_BLOCK // (_RG * _UNROLL)


def _elementwise(x, t):
    u = jnp.exp(-jnp.abs(x))
    w = 1.0 + u
    r = 1.0 / w
    ur = u * r
    p = jnp.where(x >= 0.0, r, ur)
    log1p_u = jnp.log(w)
    ce = jnp.maximum(x, 0.0) - x * t + log1p_u
    den_v = p + t
    ptv = p * t
    ompt = den_v - (ptv + ptv)
    alpha_t = (1.0 - _ALPHA) - (1.0 - 2.0 * _ALPHA) * t
    focal_el = alpha_t * ce * (ompt * ompt)
    return focal_el, den_v, ptv


def _loss_kernel(nb_ref, x_ref, t_ref, out_ref, accp_ref, accd_ref, accf_ref):
    step = pl.program_id(0)

    @pl.when(step == 0)
    def _init():
        accp_ref[...] = jnp.zeros((_RG, _NUM_MASKS), jnp.float32)
        accd_ref[...] = jnp.zeros((_RG, _NUM_MASKS), jnp.float32)
        accf_ref[...] = jnp.zeros((_RG, _NUM_MASKS), jnp.float32)

    def _chunk(j, carry):
        ap, ad, af = carry
        for k in range(_UNROLL):
            r0 = j * (_RG * _UNROLL) + k * _RG
            x = x_ref[pl.ds(r0, _RG), :]
            t = t_ref[pl.ds(r0, _RG), :]
            f_v, den_v, ptv = _elementwise(x, t)
            af = af + f_v
            ad = ad + den_v
            ap = ap + ptv
        return ap, ad, af

    zero = jnp.zeros((_RG, _NUM_MASKS), jnp.float32)
    ap, ad, af = lax.fori_loop(0, _NCHUNK, _chunk, (zero, zero, zero))
    accp_ref[...] += ap
    accd_ref[...] += ad
    accf_ref[...] += af

    @pl.when(step == _GRID - 1)
    def _finish():
        s_pt = jnp.sum(accp_ref[...], axis=0)
        s_den = jnp.sum(accd_ref[...], axis=0)
        dice_rows = 1.0 - (2.0 * s_pt + 1.0) / (s_den + 1.0)
        inv_nb = 1.0 / (nb_ref[0] + 1e-06)
        dice = jnp.sum(dice_rows) * inv_nb
        focal = jnp.sum(accf_ref[...]) * (inv_nb / _N_POINTS)
        out_ref[0] = dice + focal
        out_ref[1] = dice
        out_ref[2] = focal


def _loss(nb, xt, tt):
    return pl.pallas_call(
        _loss_kernel,
        grid=(_GRID,),
        in_specs=[
            pl.BlockSpec(memory_space=pltpu.SMEM),
            pl.BlockSpec((_PT_BLOCK, _NUM_MASKS), lambda i: (i, 0)),
            pl.BlockSpec((_PT_BLOCK, _NUM_MASKS), lambda i: (i, 0)),
        ],
        out_specs=pl.BlockSpec(memory_space=pltpu.SMEM),
        out_shape=jax.ShapeDtypeStruct((3,), jnp.float32),
        scratch_shapes=[
            pltpu.VMEM((_RG, _NUM_MASKS), jnp.float32),
            pltpu.VMEM((_RG, _NUM_MASKS), jnp.float32),
            pltpu.VMEM((_RG, _NUM_MASKS), jnp.float32),
        ],
    )(nb, xt, tt)


def kernel(mask_logits_pred, inst_mask_gt, num_boxes):
    nb = jnp.asarray(num_boxes, dtype=jnp.float32).reshape((1,))
    out = _loss(nb, mask_logits_pred.T, inst_mask_gt.T)
    return (out[0], out[1], out[2])


# speedup vs baseline: 1.0458x; 1.0458x over previous
import jax
import jax.numpy as jnp
from jax import lax
from jax.experimental import pallas as pl
from jax.experimental.pallas import tpu as pltpu

_NUM_MASKS = 512
_N_POINTS = 20000
_ALPHA = 0.25

_PT_BLOCK = 2000
_GRID = _N_POINTS // _PT_BLOCK
_RG = 8
_UNROLL = 10
_NCHUNK = _PT_BLOCK // (_RG * _UNROLL)


def _elementwise(x, t):
    a = jnp.abs(x)
    u = jnp.exp(-a)
    w = 1.0 + u
    r = 1.0 / w
    ur = u * r
    xpos = x >= 0.0
    p = jnp.where(xpos, r, ur)
    logw = jnp.log(w)
    tb = t > 0.5
    ypos = xpos != tb
    sig_y = jnp.where(ypos, r, ur)
    ce = jnp.where(ypos, a, 0.0) + logw
    alpha_t = jnp.where(tb, _ALPHA, 1.0 - _ALPHA)
    focal_el = (alpha_t * ce) * (sig_y * sig_y)
    den_v = p + t
    ptv = jnp.where(tb, p, 0.0)
    return focal_el, den_v, ptv


def _loss_kernel(nb_ref, x_ref, t_ref, out_ref, accp_ref, accd_ref, accf_ref):
    step = pl.program_id(0)

    @pl.when(step == 0)
    def _init():
        accp_ref[...] = jnp.zeros((_RG, _NUM_MASKS), jnp.float32)
        accd_ref[...] = jnp.zeros((_RG, _NUM_MASKS), jnp.float32)
        accf_ref[...] = jnp.zeros((_RG, _NUM_MASKS), jnp.float32)

    def _chunk(j, carry):
        ap, ad, af = carry
        for k in range(_UNROLL):
            r0 = j * (_RG * _UNROLL) + k * _RG
            x = x_ref[pl.ds(r0, _RG), :]
            t = t_ref[pl.ds(r0, _RG), :]
            f_v, den_v, ptv = _elementwise(x, t)
            af = af + f_v
            ad = ad + den_v
            ap = ap + ptv
        return ap, ad, af

    zero = jnp.zeros((_RG, _NUM_MASKS), jnp.float32)
    ap, ad, af = lax.fori_loop(0, _NCHUNK, _chunk, (zero, zero, zero))
    accp_ref[...] += ap
    accd_ref[...] += ad
    accf_ref[...] += af

    @pl.when(step == _GRID - 1)
    def _finish():
        s_pt = jnp.sum(accp_ref[...], axis=0)
        s_den = jnp.sum(accd_ref[...], axis=0)
        dice_rows = 1.0 - (2.0 * s_pt + 1.0) / (s_den + 1.0)
        inv_nb = 1.0 / (nb_ref[0] + 1e-06)
        dice = jnp.sum(dice_rows) * inv_nb
        focal = jnp.sum(accf_ref[...]) * (inv_nb / _N_POINTS)
        out_ref[0] = dice + focal
        out_ref[1] = dice
        out_ref[2] = focal


def _loss(nb, xt, tt):
    return pl.pallas_call(
        _loss_kernel,
        grid=(_GRID,),
        in_specs=[
            pl.BlockSpec(memory_space=pltpu.SMEM),
            pl.BlockSpec((_PT_BLOCK, _NUM_MASKS), lambda i: (i, 0)),
            pl.BlockSpec((_PT_BLOCK, _NUM_MASKS), lambda i: (i, 0)),
        ],
        out_specs=pl.BlockSpec(memory_space=pltpu.SMEM),
        out_shape=jax.ShapeDtypeStruct((3,), jnp.float32),
        scratch_shapes=[
            pltpu.VMEM((_RG, _NUM_MASKS), jnp.float32),
            pltpu.VMEM((_RG, _NUM_MASKS), jnp.float32),
            pltpu.VMEM((_RG, _NUM_MASKS), jnp.float32),
        ],
    )(nb, xt, tt)


def kernel(mask_logits_pred, inst_mask_gt, num_boxes):
    nb = jnp.asarray(num_boxes, dtype=jnp.float32).reshape((1,))
    out = _loss(nb, mask_logits_pred.T, inst_mask_gt.T)
    return (out[0], out[1], out[2])


# speedup vs baseline: 1.0897x; 1.0420x over previous
import jax
import jax.numpy as jnp
from jax import lax
from jax.experimental import pallas as pl
from jax.experimental.pallas import tpu as pltpu

_NUM_MASKS = 512
_N_POINTS = 20000
_ALPHA = 0.25

_PT_BLOCK = 2000
_GRID = _N_POINTS // _PT_BLOCK
_RG = 8
_UNROLL = 25
_NCHUNK = _PT_BLOCK // (_RG * _UNROLL)


def _elementwise(x, t):
    a = jnp.abs(x)
    u = jnp.exp(-a)
    w = 1.0 + u
    r = 1.0 / w
    ur = u * r
    xpos = x >= 0.0
    p = jnp.where(xpos, r, ur)
    logw = jnp.log(w)
    tb = t > 0.5
    ypos = xpos != tb
    sig_y = jnp.where(ypos, r, ur)
    ce = jnp.where(ypos, a, 0.0) + logw
    alpha_t = jnp.where(tb, _ALPHA, 1.0 - _ALPHA)
    focal_el = (alpha_t * ce) * (sig_y * sig_y)
    den_v = p + t
    ptv = jnp.where(tb, p, 0.0)
    return focal_el, den_v, ptv


def _loss_kernel(nb_ref, x_ref, t_ref, out_ref, accp_ref, accd_ref, accf_ref):
    step = pl.program_id(0)

    @pl.when(step == 0)
    def _init():
        accp_ref[...] = jnp.zeros((_RG, _NUM_MASKS), jnp.float32)
        accd_ref[...] = jnp.zeros((_RG, _NUM_MASKS), jnp.float32)
        accf_ref[...] = jnp.zeros((_RG, _NUM_MASKS), jnp.float32)

    def _chunk(j, carry):
        ap, ad, af = carry
        for k in range(_UNROLL):
            r0 = j * (_RG * _UNROLL) + k * _RG
            x = x_ref[pl.ds(r0, _RG), :]
            t = t_ref[pl.ds(r0, _RG), :]
            f_v, den_v, ptv = _elementwise(x, t)
            af = af + f_v
            ad = ad + den_v
            ap = ap + ptv
        return ap, ad, af

    zero = jnp.zeros((_RG, _NUM_MASKS), jnp.float32)
    ap, ad, af = lax.fori_loop(0, _NCHUNK, _chunk, (zero, zero, zero))
    accp_ref[...] += ap
    accd_ref[...] += ad
    accf_ref[...] += af

    @pl.when(step == _GRID - 1)
    def _finish():
        s_pt = jnp.sum(accp_ref[...], axis=0)
        s_den = jnp.sum(accd_ref[...], axis=0)
        dice_rows = 1.0 - (2.0 * s_pt + 1.0) / (s_den + 1.0)
        inv_nb = 1.0 / (nb_ref[0] + 1e-06)
        dice = jnp.sum(dice_rows) * inv_nb
        focal = jnp.sum(accf_ref[...]) * (inv_nb / _N_POINTS)
        out_ref[0] = dice + focal
        out_ref[1] = dice
        out_ref[2] = focal


def _loss(nb, xt, tt):
    return pl.pallas_call(
        _loss_kernel,
        grid=(_GRID,),
        in_specs=[
            pl.BlockSpec(memory_space=pltpu.SMEM),
            pl.BlockSpec((_PT_BLOCK, _NUM_MASKS), lambda i: (i, 0)),
            pl.BlockSpec((_PT_BLOCK, _NUM_MASKS), lambda i: (i, 0)),
        ],
        out_specs=pl.BlockSpec(memory_space=pltpu.SMEM),
        out_shape=jax.ShapeDtypeStruct((3,), jnp.float32),
        scratch_shapes=[
            pltpu.VMEM((_RG, _NUM_MASKS), jnp.float32),
            pltpu.VMEM((_RG, _NUM_MASKS), jnp.float32),
            pltpu.VMEM((_RG, _NUM_MASKS), jnp.float32),
        ],
    )(nb, xt, tt)


def kernel(mask_logits_pred, inst_mask_gt, num_boxes):
    nb = jnp.asarray(num_boxes, dtype=jnp.float32).reshape((1,))
    out = _loss(nb, mask_logits_pred.T, inst_mask_gt.T)
    return (out[0], out[1], out[2])


# speedup vs baseline: 1.0934x; 1.0034x over previous
import jax
import jax.numpy as jnp
from jax import lax
from jax.experimental import pallas as pl
from jax.experimental.pallas import tpu as pltpu

_NUM_MASKS = 512
_N_POINTS = 20000
_ALPHA = 0.25

_PT_BLOCK = 2000
_GRID = _N_POINTS // _PT_BLOCK
_RG = 8
_UNROLL = 50
_NCHUNK = _PT_BLOCK // (_RG * _UNROLL)


def _elementwise(x, t):
    a = jnp.abs(x)
    u = jnp.exp(-a)
    w = 1.0 + u
    r = 1.0 / w
    ur = u * r
    xpos = x >= 0.0
    p = jnp.where(xpos, r, ur)
    logw = jnp.log(w)
    tb = t > 0.5
    ypos = xpos != tb
    sig_y = jnp.where(ypos, r, ur)
    ce = jnp.where(ypos, a, 0.0) + logw
    alpha_t = jnp.where(tb, _ALPHA, 1.0 - _ALPHA)
    focal_el = (alpha_t * ce) * (sig_y * sig_y)
    den_v = p + t
    ptv = jnp.where(tb, p, 0.0)
    return focal_el, den_v, ptv


def _loss_kernel(nb_ref, x_ref, t_ref, out_ref, accp_ref, accd_ref, accf_ref):
    step = pl.program_id(0)

    @pl.when(step == 0)
    def _init():
        accp_ref[...] = jnp.zeros((_RG, _NUM_MASKS), jnp.float32)
        accd_ref[...] = jnp.zeros((_RG, _NUM_MASKS), jnp.float32)
        accf_ref[...] = jnp.zeros((_RG, _NUM_MASKS), jnp.float32)

    def _chunk(j, carry):
        ap, ad, af = carry
        for k in range(_UNROLL):
            r0 = j * (_RG * _UNROLL) + k * _RG
            x = x_ref[pl.ds(r0, _RG), :]
            t = t_ref[pl.ds(r0, _RG), :]
            f_v, den_v, ptv = _elementwise(x, t)
            af = af + f_v
            ad = ad + den_v
            ap = ap + ptv
        return ap, ad, af

    zero = jnp.zeros((_RG, _NUM_MASKS), jnp.float32)
    ap, ad, af = lax.fori_loop(0, _NCHUNK, _chunk, (zero, zero, zero))
    accp_ref[...] += ap
    accd_ref[...] += ad
    accf_ref[...] += af

    @pl.when(step == _GRID - 1)
    def _finish():
        s_pt = jnp.sum(accp_ref[...], axis=0)
        s_den = jnp.sum(accd_ref[...], axis=0)
        dice_rows = 1.0 - (2.0 * s_pt + 1.0) / (s_den + 1.0)
        inv_nb = 1.0 / (nb_ref[0] + 1e-06)
        dice = jnp.sum(dice_rows) * inv_nb
        focal = jnp.sum(accf_ref[...]) * (inv_nb / _N_POINTS)
        out_ref[0] = dice + focal
        out_ref[1] = dice
        out_ref[2] = focal


def _loss(nb, xt, tt):
    return pl.pallas_call(
        _loss_kernel,
        grid=(_GRID,),
        in_specs=[
            pl.BlockSpec(memory_space=pltpu.SMEM),
            pl.BlockSpec((_PT_BLOCK, _NUM_MASKS), lambda i: (i, 0)),
            pl.BlockSpec((_PT_BLOCK, _NUM_MASKS), lambda i: (i, 0)),
        ],
        out_specs=pl.BlockSpec(memory_space=pltpu.SMEM),
        out_shape=jax.ShapeDtypeStruct((3,), jnp.float32),
        scratch_shapes=[
            pltpu.VMEM((_RG, _NUM_MASKS), jnp.float32),
            pltpu.VMEM((_RG, _NUM_MASKS), jnp.float32),
            pltpu.VMEM((_RG, _NUM_MASKS), jnp.float32),
        ],
    )(nb, xt, tt)


def kernel(mask_logits_pred, inst_mask_gt, num_boxes):
    nb = jnp.asarray(num_boxes, dtype=jnp.float32).reshape((1,))
    out = _loss(nb, mask_logits_pred.T, inst_mask_gt.T)
    return (out[0], out[1], out[2])
